# baseline (speedup 1.0000x reference)
import jax
import jax.numpy as jnp
from jax import lax
from jax.experimental import pallas as pl
from jax.experimental.pallas import tpu as pltpu
from jax.experimental.pallas import tpu_sc as plsc

B, S, D = 4, 4096, 1024
NC, NS = 2, 16
NW = NC * NS
S_PER_W = S // NW
RCHUNK = 8
NCHUNK = S_PER_W // RCHUNK


NSTAGE = 3


def _sc_body(x_hbm, e_hbm, o_hbm, *refs):
    xbuf = list(refs[0:NSTAGE])
    ebuf = list(refs[NSTAGE:2 * NSTAGE])
    xsem = list(refs[2 * NSTAGE:3 * NSTAGE])
    esem = list(refs[3 * NSTAGE:4 * NSTAGE])
    osem = list(refs[4 * NSTAGE:5 * NSTAGE])

    cid = lax.axis_index("c")
    sid = lax.axis_index("s")
    wid = sid * NC + cid
    s_base = wid * S_PER_W

    def load_x(c):
        return pltpu.async_copy(
            x_hbm.at[:, pl.ds(s_base + c * RCHUNK, RCHUNK), :],
            xbuf[c % NSTAGE], xsem[c % NSTAGE])

    def load_e(c):
        return pltpu.async_copy(
            e_hbm.at[pl.ds(s_base + c * RCHUNK, RCHUNK), :],
            ebuf[c % NSTAGE], esem[c % NSTAGE])

    def store_o(c):
        return pltpu.async_copy(
            xbuf[c % NSTAGE],
            o_hbm.at[:, pl.ds(s_base + c * RCHUNK, RCHUNK), :],
            osem[c % NSTAGE])

    x_pend = {0: load_x(0)}
    e_pend = {0: load_e(0)}
    o_pend = {}

    for c in range(NCHUNK):
        p = c % NSTAGE
        if c + 1 < NCHUNK:
            if c + 1 - NSTAGE in o_pend:
                o_pend.pop(c + 1 - NSTAGE).wait()
            e_pend[c + 1] = load_e(c + 1)
            x_pend[c + 1] = load_x(c + 1)
        e_pend.pop(c).wait()
        x_pend.pop(c).wait()

        eb = ebuf[p]
        xb = xbuf[p]

        @plsc.parallel_loop(0, D, step=16)
        def add(i):
            sl = pl.ds(i, 16)
            for r in range(RCHUNK):
                ev = eb[r, sl]
                for b in range(B):
                    xb[b, r, sl] = xb[b, r, sl] + ev

        o_pend[c] = store_o(c)

    for c in sorted(o_pend):
        o_pend.pop(c).wait()


@jax.jit
def _sc_add(x, e):
    mesh = plsc.VectorSubcoreMesh(core_axis_name="c", subcore_axis_name="s")
    return pl.kernel(
        _sc_body,
        mesh=mesh,
        out_type=jax.ShapeDtypeStruct((B, S, D), jnp.float32),
        scratch_types=(
            [pltpu.VMEM((B, RCHUNK, D), jnp.float32) for _ in range(NSTAGE)]
            + [pltpu.VMEM((RCHUNK, D), jnp.float32) for _ in range(NSTAGE)]
            + [pltpu.SemaphoreType.DMA for _ in range(3 * NSTAGE)]
        ),
    )(x, e)


def kernel(x, emb_weight):
    return _sc_add(x, emb_weight)


# speedup vs baseline: 1.0162x; 1.0162x over previous
import jax
import jax.numpy as jnp
from jax import lax
from jax.experimental import pallas as pl
from jax.experimental.pallas import tpu as pltpu
from jax.experimental.pallas import tpu_sc as plsc

B, S, D = 4, 4096, 1024
NC, NS = 2, 16
NW = NC * NS
S_PER_W = S // NW
RCHUNK = 8
NCHUNK = S_PER_W // RCHUNK


NSTAGE = 3


def _sc_body(x_hbm, e_hbm, o_hbm, *refs):
    xbuf = list(refs[0:NSTAGE])
    ebuf = list(refs[NSTAGE:2 * NSTAGE])
    xsem = list(refs[2 * NSTAGE:3 * NSTAGE])
    esem = list(refs[3 * NSTAGE:4 * NSTAGE])
    osem = list(refs[4 * NSTAGE:5 * NSTAGE])

    cid = lax.axis_index("c")
    sid = lax.axis_index("s")
    wid = sid * NC + cid
    s_base = wid * S_PER_W

    def load_x(c):
        return pltpu.async_copy(
            x_hbm.at[:, pl.ds(s_base + c * RCHUNK, RCHUNK), :],
            xbuf[c % NSTAGE], xsem[c % NSTAGE])

    def load_e(c):
        return pltpu.async_copy(
            e_hbm.at[pl.ds(s_base + c * RCHUNK, RCHUNK), :],
            ebuf[c % NSTAGE], esem[c % NSTAGE])

    def store_o(c):
        return pltpu.async_copy(
            xbuf[c % NSTAGE],
            o_hbm.at[:, pl.ds(s_base + c * RCHUNK, RCHUNK), :],
            osem[c % NSTAGE])

    x_pend = {c: load_x(c) for c in range(NSTAGE - 1)}
    e_pend = {c: load_e(c) for c in range(NSTAGE - 1)}
    o_pend = {}

    for c in range(NCHUNK):
        p = c % NSTAGE
        cn = c + NSTAGE - 1
        if cn < NCHUNK:
            if cn - NSTAGE in o_pend:
                o_pend.pop(cn - NSTAGE).wait()
            e_pend[cn] = load_e(cn)
            x_pend[cn] = load_x(cn)
        e_pend.pop(c).wait()
        x_pend.pop(c).wait()

        eb = ebuf[p]
        xb = xbuf[p]

        @plsc.parallel_loop(0, D, step=16)
        def add(i):
            sl = pl.ds(i, 16)
            for r in range(RCHUNK):
                ev = eb[r, sl]
                for b in range(B):
                    xb[b, r, sl] = xb[b, r, sl] + ev

        o_pend[c] = store_o(c)

    for c in sorted(o_pend):
        o_pend.pop(c).wait()


@jax.jit
def _sc_add(x, e):
    mesh = plsc.VectorSubcoreMesh(core_axis_name="c", subcore_axis_name="s")
    return pl.kernel(
        _sc_body,
        mesh=mesh,
        out_type=jax.ShapeDtypeStruct((B, S, D), jnp.float32),
        scratch_types=(
            [pltpu.VMEM((B, RCHUNK, D), jnp.float32) for _ in range(NSTAGE)]
            + [pltpu.VMEM((RCHUNK, D), jnp.float32) for _ in range(NSTAGE)]
            + [pltpu.SemaphoreType.DMA for _ in range(3 * NSTAGE)]
        ),
    )(x, e)


def kernel(x, emb_weight):
    return _sc_add(x, emb_weight)


# speedup vs baseline: 1.0329x; 1.0164x over previous
import jax
import jax.numpy as jnp
from jax import lax
from jax.experimental import pallas as pl
from jax.experimental.pallas import tpu as pltpu
from jax.experimental.pallas import tpu_sc as plsc

B, S, D = 4, 4096, 1024
NC, NS = 2, 16
NW = NC * NS
S_PER_W = S // NW
RCHUNK = 4
NCHUNK = S_PER_W // RCHUNK


NSTAGE = 6


def _sc_body(x_hbm, e_hbm, o_hbm, *refs):
    xbuf = list(refs[0:NSTAGE])
    ebuf = list(refs[NSTAGE:2 * NSTAGE])
    xsem = list(refs[2 * NSTAGE:3 * NSTAGE])
    esem = list(refs[3 * NSTAGE:4 * NSTAGE])
    osem = list(refs[4 * NSTAGE:5 * NSTAGE])

    cid = lax.axis_index("c")
    sid = lax.axis_index("s")
    wid = sid * NC + cid
    s_base = wid * S_PER_W

    def load_x(c):
        return pltpu.async_copy(
            x_hbm.at[:, pl.ds(s_base + c * RCHUNK, RCHUNK), :],
            xbuf[c % NSTAGE], xsem[c % NSTAGE])

    def load_e(c):
        return pltpu.async_copy(
            e_hbm.at[pl.ds(s_base + c * RCHUNK, RCHUNK), :],
            ebuf[c % NSTAGE], esem[c % NSTAGE])

    def store_o(c):
        return pltpu.async_copy(
            xbuf[c % NSTAGE],
            o_hbm.at[:, pl.ds(s_base + c * RCHUNK, RCHUNK), :],
            osem[c % NSTAGE])

    x_pend = {c: load_x(c) for c in range(NSTAGE - 1)}
    e_pend = {c: load_e(c) for c in range(NSTAGE - 1)}
    o_pend = {}

    for c in range(NCHUNK):
        p = c % NSTAGE
        cn = c + NSTAGE - 1
        if cn < NCHUNK:
            if cn - NSTAGE in o_pend:
                o_pend.pop(cn - NSTAGE).wait()
            e_pend[cn] = load_e(cn)
            x_pend[cn] = load_x(cn)
        e_pend.pop(c).wait()
        x_pend.pop(c).wait()

        eb = ebuf[p]
        xb = xbuf[p]

        @plsc.parallel_loop(0, D, step=16)
        def add(i):
            sl = pl.ds(i, 16)
            for r in range(RCHUNK):
                ev = eb[r, sl]
                for b in range(B):
                    xb[b, r, sl] = xb[b, r, sl] + ev

        o_pend[c] = store_o(c)

    for c in sorted(o_pend):
        o_pend.pop(c).wait()


@jax.jit
def _sc_add(x, e):
    mesh = plsc.VectorSubcoreMesh(core_axis_name="c", subcore_axis_name="s")
    return pl.kernel(
        _sc_body,
        mesh=mesh,
        out_type=jax.ShapeDtypeStruct((B, S, D), jnp.float32),
        scratch_types=(
            [pltpu.VMEM((B, RCHUNK, D), jnp.float32) for _ in range(NSTAGE)]
            + [pltpu.VMEM((RCHUNK, D), jnp.float32) for _ in range(NSTAGE)]
            + [pltpu.SemaphoreType.DMA for _ in range(3 * NSTAGE)]
        ),
    )(x, e)


def kernel(x, emb_weight):
    return _sc_add(x, emb_weight)
